# baseline (speedup 1.0000x reference)
import functools

import jax
import jax.numpy as jnp
from jax import lax
from jax.experimental import pallas as pl
from jax.experimental.pallas import tpu as pltpu
from jax.experimental.pallas import tpu_sc as plsc

EMBED_DIM = 768
HIDDEN_DIM = 3072
NUM_EXPERTS = 8
TOP_K = 2

BT = 256
T_TOKENS = 2048
NPAIR = T_TOKENS * TOP_K
NG = NPAIR // BT + NUM_EXPERTS
NP_ROWS = NG * BT

NW = 32
GC = 48
RPW = NP_ROWS // NW
NCH = RPW // GC
NBUF = 3
TPW = T_TOKENS // NW


def _router_kernel(x_ref, wr_ref, br_ref, eidx_ref, pw_ref, xc_ref):
    xc_ref[...] = x_ref[...]
    logits = jnp.dot(x_ref[...], wr_ref[...],
                     preferred_element_type=jnp.float32) + br_ref[...]
    lane = lax.broadcasted_iota(jnp.int32, logits.shape, 1)
    m1 = jnp.max(logits, axis=1, keepdims=True)
    i1 = jnp.min(jnp.where(logits == m1, lane, NUM_EXPERTS), axis=1,
                 keepdims=True)
    l2 = jnp.where(lane == i1, -jnp.inf, logits)
    m2 = jnp.max(l2, axis=1, keepdims=True)
    i2 = jnp.min(jnp.where(l2 == m2, lane, NUM_EXPERTS), axis=1, keepdims=True)
    p2 = 1.0 / (1.0 + jnp.exp(m1 - m2))
    p1 = 1.0 - p2
    eidx_ref[...] = jnp.concatenate([i1, i2], axis=1)
    pw_ref[...] = jnp.concatenate([p1, p2], axis=1)


def _ffn_kernel(blk_e_ref, xg_ref, w_ref, w1_ref, b1_ref, w2_ref, b2_ref,
                y_ref):
    del blk_e_ref
    h = jnp.dot(xg_ref[...], w1_ref[0], preferred_element_type=jnp.float32)
    h = jnp.maximum(h + b1_ref[0], 0.0)
    y = jnp.dot(h, w2_ref[0], preferred_element_type=jnp.float32) + b2_ref[0]
    y_ref[...] = w_ref[0] * y


def _dispatch_kernel(x_hbm, tok_hbm, xg_hbm, i0, i1, i2, i3, r0, r1, r2,
                     gsem, wsem):
    wid = lax.axis_index("s") * 2 + lax.axis_index("c")
    base = wid * RPW
    idxs = (i0, i1, i2, i3)
    bufs = (r0, r1, r2)
    for ci in range(NCH):
        pltpu.sync_copy(tok_hbm.at[wid, ci], idxs[ci])
    g, w = [None] * NCH, [None] * NCH
    for b in range(min(NBUF, NCH)):
        g[b] = pltpu.async_copy(x_hbm.at[idxs[b]], bufs[b], gsem)
    for ci in range(NCH):
        g[ci].wait()
        w[ci] = pltpu.async_copy(bufs[ci % NBUF],
                                 xg_hbm.at[pl.ds(base + ci * GC, GC)], wsem)
        nxt = ci + NBUF
        if nxt < NCH:
            w[ci].wait()
            g[nxt] = pltpu.async_copy(x_hbm.at[idxs[nxt]],
                                      bufs[nxt % NBUF], gsem)
    for ci in range(NCH):
        if ci + NBUF >= NCH:
            w[ci].wait()


def _combine_kernel(y_hbm, pos_hbm, out_hbm, p0_v, p1_v, buf0, buf1, sem):
    wid = lax.axis_index("s") * 2 + lax.axis_index("c")
    base = wid * TPW
    pltpu.sync_copy(pos_hbm.at[wid, 0], p0_v)
    pltpu.sync_copy(pos_hbm.at[wid, 1], p1_v)
    c0 = pltpu.async_copy(y_hbm.at[p0_v], buf0, sem)
    c1 = pltpu.async_copy(y_hbm.at[p1_v], buf1, sem)
    c0.wait()
    c1.wait()

    def row(r, carry):
        for j in range(EMBED_DIM // 16):
            sl = pl.ds(j * 16, 16)
            buf0[r, sl] = buf0[r, sl] + buf1[r, sl]
        return carry

    lax.fori_loop(0, TPW, row, 0)
    pltpu.sync_copy(buf0, out_hbm.at[pl.ds(base, TPW)])


def kernel(x, Wr, br, W1, b1, W2, b2):
    batch, seq, d = x.shape
    x_flat = x.reshape(-1, d)
    T = x_flat.shape[0]

    eidx, pw, x_c = pl.pallas_call(
        _router_kernel,
        grid=(1,),
        in_specs=[
            pl.BlockSpec((T, d), lambda i: (0, 0)),
            pl.BlockSpec((d, NUM_EXPERTS), lambda i: (0, 0)),
            pl.BlockSpec((1, NUM_EXPERTS), lambda i: (0, 0)),
        ],
        out_specs=[
            pl.BlockSpec((T, TOP_K), lambda i: (0, 0)),
            pl.BlockSpec((T, TOP_K), lambda i: (0, 0)),
            pl.BlockSpec((T, d), lambda i: (0, 0)),
        ],
        out_shape=[
            jax.ShapeDtypeStruct((T, TOP_K), jnp.int32),
            jax.ShapeDtypeStruct((T, TOP_K), jnp.float32),
            jax.ShapeDtypeStruct((T, d), jnp.float32),
        ],
    )(x_flat, Wr, br.reshape(1, NUM_EXPERTS))

    tok_pad, w_pad, blk_e, pos = _plan(eidx, pw)

    mesh = plsc.VectorSubcoreMesh(core_axis_name="c", subcore_axis_name="s")

    tok3 = tok_pad.reshape(NW, NCH, GC)
    xg = pl.kernel(
        _dispatch_kernel,
        mesh=mesh,
        out_type=jax.ShapeDtypeStruct((NP_ROWS, d), jnp.float32),
        scratch_types=[
            pltpu.VMEM((GC,), jnp.int32),
            pltpu.VMEM((GC,), jnp.int32),
            pltpu.VMEM((GC,), jnp.int32),
            pltpu.VMEM((GC,), jnp.int32),
            pltpu.VMEM((GC, d), jnp.float32),
            pltpu.VMEM((GC, d), jnp.float32),
            pltpu.VMEM((GC, d), jnp.float32),
            pltpu.SemaphoreType.DMA,
            pltpu.SemaphoreType.DMA,
        ],
    )(x_c, tok3)

    y = pl.pallas_call(
        _ffn_kernel,
        grid_spec=pltpu.PrefetchScalarGridSpec(
            num_scalar_prefetch=1,
            grid=(NG,),
            in_specs=[
                pl.BlockSpec((BT, d), lambda g, be: (g, 0)),
                pl.BlockSpec((1, BT, 1), lambda g, be: (g, 0, 0)),
                pl.BlockSpec((1, d, HIDDEN_DIM), lambda g, be: (be[g], 0, 0)),
                pl.BlockSpec((1, 1, HIDDEN_DIM), lambda g, be: (be[g], 0, 0)),
                pl.BlockSpec((1, HIDDEN_DIM, d), lambda g, be: (be[g], 0, 0)),
                pl.BlockSpec((1, 1, d), lambda g, be: (be[g], 0, 0)),
            ],
            out_specs=pl.BlockSpec((BT, d), lambda g, be: (g, 0)),
        ),
        out_shape=jax.ShapeDtypeStruct((NP_ROWS, d), jnp.float32),
    )(blk_e, xg, w_pad.reshape(NG, BT, 1), W1,
      b1.reshape(NUM_EXPERTS, 1, HIDDEN_DIM), W2,
      b2.reshape(NUM_EXPERTS, 1, EMBED_DIM))

    out = pl.kernel(
        _combine_kernel,
        mesh=mesh,
        out_type=jax.ShapeDtypeStruct((T, d), jnp.float32),
        scratch_types=[
            pltpu.VMEM((TPW,), jnp.int32),
            pltpu.VMEM((TPW,), jnp.int32),
            pltpu.VMEM((TPW, d), jnp.float32),
            pltpu.VMEM((TPW, d), jnp.float32),
            pltpu.SemaphoreType.DMA,
        ],
    )(y, pos)

    return out.reshape(batch, seq, d)


def _plan(eidx, pw):
    T = T_TOKENS
    e_flat = eidx.reshape(-1)
    w_flat = pw.reshape(-1)
    oh = (e_flat[:, None] == jnp.arange(NUM_EXPERTS)[None, :]).astype(jnp.int32)
    cnt_incl = jnp.cumsum(oh, axis=0)
    cnt_before = jnp.take_along_axis(cnt_incl, e_flat[:, None], 1)[:, 0] - 1
    counts = cnt_incl[-1]
    nb = (counts + BT - 1) // BT
    bounds = jnp.concatenate([jnp.zeros((1,), jnp.int32),
                              jnp.cumsum(nb)[:-1]]).astype(jnp.int32)
    pad_base = BT * bounds
    padpos = pad_base[e_flat] + cnt_before
    tok_pad = jnp.zeros((NP_ROWS,), jnp.int32).at[padpos].set(
        jnp.arange(NPAIR, dtype=jnp.int32) // TOP_K, unique_indices=True)
    w_pad = jnp.zeros((NP_ROWS,), jnp.float32).at[padpos].set(
        w_flat, unique_indices=True)
    blk_e = jnp.clip(
        jnp.searchsorted(bounds, jnp.arange(NG, dtype=jnp.int32),
                         side='right').astype(jnp.int32) - 1,
        0, NUM_EXPERTS - 1)
    pos = padpos.reshape(NW, TPW, TOP_K).transpose(0, 2, 1)
    return tok_pad, w_pad, blk_e, pos


# speedup vs baseline: 1.5515x; 1.5515x over previous
import functools

import jax
import jax.numpy as jnp
from jax import lax
from jax.experimental import pallas as pl
from jax.experimental.pallas import tpu as pltpu
from jax.experimental.pallas import tpu_sc as plsc

EMBED_DIM = 768
HIDDEN_DIM = 3072
NUM_EXPERTS = 8
TOP_K = 2

BT = 256
T_TOKENS = 2048
NPAIR = T_TOKENS * TOP_K
NG = NPAIR // BT + NUM_EXPERTS
NP_ROWS = NG * BT

NW = 32
GC = 48
RPW = NP_ROWS // NW
NCH = RPW // GC
NBUF = 3
TPW = T_TOKENS // NW


def _router_kernel(x_ref, wr_ref, br_ref, eidx_ref, pw_ref):
    logits = jnp.dot(x_ref[...], wr_ref[...],
                     preferred_element_type=jnp.float32) + br_ref[...]
    lane = lax.broadcasted_iota(jnp.int32, logits.shape, 1)
    m1 = jnp.max(logits, axis=1, keepdims=True)
    i1 = jnp.min(jnp.where(logits == m1, lane, NUM_EXPERTS), axis=1,
                 keepdims=True)
    l2 = jnp.where(lane == i1, -jnp.inf, logits)
    m2 = jnp.max(l2, axis=1, keepdims=True)
    i2 = jnp.min(jnp.where(l2 == m2, lane, NUM_EXPERTS), axis=1, keepdims=True)
    p2 = 1.0 / (1.0 + jnp.exp(m1 - m2))
    p1 = 1.0 - p2
    eidx_ref[...] = jnp.concatenate([i1, i2], axis=1)
    pw_ref[...] = jnp.concatenate([p1, p2], axis=1)


def _ffn_kernel(blk_e_ref, xg_ref, w_ref, w1_ref, b1_ref, w2_ref, b2_ref,
                y_ref):
    del blk_e_ref
    h = jnp.dot(xg_ref[...], w1_ref[0], preferred_element_type=jnp.float32)
    h = jnp.maximum(h + b1_ref[0], 0.0)
    y = jnp.dot(h, w2_ref[0], preferred_element_type=jnp.float32) + b2_ref[0]
    y_ref[...] = w_ref[0] * y


def _dispatch_kernel(x_hbm, pos_hbm, xg_hbm, p0_v, p1_v, xbuf, gsem, wsem):
    wid = lax.axis_index("s") * 2 + lax.axis_index("c")
    base = wid * TPW
    pltpu.sync_copy(pos_hbm.at[wid, 0], p0_v)
    pltpu.sync_copy(pos_hbm.at[wid, 1], p1_v)
    pltpu.async_copy(x_hbm.at[pl.ds(base, TPW)], xbuf, gsem).wait()
    s0 = pltpu.async_copy(xbuf, xg_hbm.at[p0_v], wsem)
    s1 = pltpu.async_copy(xbuf, xg_hbm.at[p1_v], wsem)
    s0.wait()
    s1.wait()


def _combine_kernel(y_hbm, pos_hbm, out_hbm, p0_v, p1_v, buf0, buf1, sem):
    wid = lax.axis_index("s") * 2 + lax.axis_index("c")
    base = wid * TPW
    pltpu.sync_copy(pos_hbm.at[wid, 0], p0_v)
    pltpu.sync_copy(pos_hbm.at[wid, 1], p1_v)
    c0 = pltpu.async_copy(y_hbm.at[p0_v], buf0, sem)
    c1 = pltpu.async_copy(y_hbm.at[p1_v], buf1, sem)
    c0.wait()
    c1.wait()

    def row(r, carry):
        for j in range(EMBED_DIM // 16):
            sl = pl.ds(j * 16, 16)
            buf0[r, sl] = buf0[r, sl] + buf1[r, sl]
        return carry

    lax.fori_loop(0, TPW, row, 0)
    pltpu.sync_copy(buf0, out_hbm.at[pl.ds(base, TPW)])


def kernel(x, Wr, br, W1, b1, W2, b2):
    batch, seq, d = x.shape
    x_flat = x.reshape(-1, d)
    T = x_flat.shape[0]

    eidx, pw = pl.pallas_call(
        _router_kernel,
        grid=(1,),
        in_specs=[
            pl.BlockSpec((T, d), lambda i: (0, 0)),
            pl.BlockSpec((d, NUM_EXPERTS), lambda i: (0, 0)),
            pl.BlockSpec((1, NUM_EXPERTS), lambda i: (0, 0)),
        ],
        out_specs=[
            pl.BlockSpec((T, TOP_K), lambda i: (0, 0)),
            pl.BlockSpec((T, TOP_K), lambda i: (0, 0)),
        ],
        out_shape=[
            jax.ShapeDtypeStruct((T, TOP_K), jnp.int32),
            jax.ShapeDtypeStruct((T, TOP_K), jnp.float32),
        ],
    )(x_flat, Wr, br.reshape(1, NUM_EXPERTS))

    tok_pad, w_pad, blk_e, pos = _plan(eidx, pw)

    mesh = plsc.VectorSubcoreMesh(core_axis_name="c", subcore_axis_name="s")

    del tok_pad
    xg = pl.kernel(
        _dispatch_kernel,
        mesh=mesh,
        out_type=jax.ShapeDtypeStruct((NP_ROWS, d), jnp.float32),
        scratch_types=[
            pltpu.VMEM((TPW,), jnp.int32),
            pltpu.VMEM((TPW,), jnp.int32),
            pltpu.VMEM((TPW, d), jnp.float32),
            pltpu.SemaphoreType.DMA,
            pltpu.SemaphoreType.DMA,
        ],
    )(x_flat, pos)

    y = pl.pallas_call(
        _ffn_kernel,
        grid_spec=pltpu.PrefetchScalarGridSpec(
            num_scalar_prefetch=1,
            grid=(NG,),
            in_specs=[
                pl.BlockSpec((BT, d), lambda g, be: (g, 0)),
                pl.BlockSpec((1, BT, 1), lambda g, be: (g, 0, 0)),
                pl.BlockSpec((1, d, HIDDEN_DIM), lambda g, be: (be[g], 0, 0)),
                pl.BlockSpec((1, 1, HIDDEN_DIM), lambda g, be: (be[g], 0, 0)),
                pl.BlockSpec((1, HIDDEN_DIM, d), lambda g, be: (be[g], 0, 0)),
                pl.BlockSpec((1, 1, d), lambda g, be: (be[g], 0, 0)),
            ],
            out_specs=pl.BlockSpec((BT, d), lambda g, be: (g, 0)),
        ),
        out_shape=jax.ShapeDtypeStruct((NP_ROWS, d), jnp.float32),
    )(blk_e, xg, w_pad.reshape(NG, BT, 1), W1,
      b1.reshape(NUM_EXPERTS, 1, HIDDEN_DIM), W2,
      b2.reshape(NUM_EXPERTS, 1, EMBED_DIM))

    out = pl.kernel(
        _combine_kernel,
        mesh=mesh,
        out_type=jax.ShapeDtypeStruct((T, d), jnp.float32),
        scratch_types=[
            pltpu.VMEM((TPW,), jnp.int32),
            pltpu.VMEM((TPW,), jnp.int32),
            pltpu.VMEM((TPW, d), jnp.float32),
            pltpu.VMEM((TPW, d), jnp.float32),
            pltpu.SemaphoreType.DMA,
        ],
    )(y, pos)

    return out.reshape(batch, seq, d)


def _plan(eidx, pw):
    T = T_TOKENS
    e_flat = eidx.reshape(-1)
    w_flat = pw.reshape(-1)
    oh = (e_flat[:, None] == jnp.arange(NUM_EXPERTS)[None, :]).astype(jnp.int32)
    cnt_incl = jnp.cumsum(oh, axis=0)
    cnt_before = jnp.take_along_axis(cnt_incl, e_flat[:, None], 1)[:, 0] - 1
    counts = cnt_incl[-1]
    nb = (counts + BT - 1) // BT
    bounds = jnp.concatenate([jnp.zeros((1,), jnp.int32),
                              jnp.cumsum(nb)[:-1]]).astype(jnp.int32)
    pad_base = BT * bounds
    padpos = pad_base[e_flat] + cnt_before
    tok_pad = jnp.zeros((NP_ROWS,), jnp.int32).at[padpos].set(
        jnp.arange(NPAIR, dtype=jnp.int32) // TOP_K, unique_indices=True)
    w_pad = jnp.zeros((NP_ROWS,), jnp.float32).at[padpos].set(
        w_flat, unique_indices=True)
    blk_e = jnp.clip(
        jnp.searchsorted(bounds, jnp.arange(NG, dtype=jnp.int32),
                         side='right').astype(jnp.int32) - 1,
        0, NUM_EXPERTS - 1)
    pos = padpos.reshape(NW, TPW, TOP_K).transpose(0, 2, 1)
    return tok_pad, w_pad, blk_e, pos


# speedup vs baseline: 1.6733x; 1.0785x over previous
import functools

import jax
import jax.numpy as jnp
from jax import lax
from jax.experimental import pallas as pl
from jax.experimental.pallas import tpu as pltpu
from jax.experimental.pallas import tpu_sc as plsc

EMBED_DIM = 768
HIDDEN_DIM = 3072
NUM_EXPERTS = 8
TOP_K = 2

BT = 128
T_TOKENS = 2048
NPAIR = T_TOKENS * TOP_K
NG = NPAIR // BT + NUM_EXPERTS
NP_ROWS = NG * BT

NW = 32
TPW = T_TOKENS // NW


def _router_kernel(x_ref, wr_ref, br_ref, eidx_ref, pw_ref):
    logits = jnp.dot(x_ref[...], wr_ref[...],
                     preferred_element_type=jnp.float32) + br_ref[...]
    lane = lax.broadcasted_iota(jnp.int32, logits.shape, 1)
    m1 = jnp.max(logits, axis=1, keepdims=True)
    i1 = jnp.min(jnp.where(logits == m1, lane, NUM_EXPERTS), axis=1,
                 keepdims=True)
    l2 = jnp.where(lane == i1, -jnp.inf, logits)
    m2 = jnp.max(l2, axis=1, keepdims=True)
    i2 = jnp.min(jnp.where(l2 == m2, lane, NUM_EXPERTS), axis=1, keepdims=True)
    p2 = 1.0 / (1.0 + jnp.exp(m1 - m2))
    p1 = 1.0 - p2
    eidx_ref[...] = jnp.concatenate([i1, i2], axis=1)
    pw_ref[...] = jnp.concatenate([p1, p2], axis=1)


def _ffn_kernel(blk_e_ref, xg_ref, w1_ref, b1_ref, w2_ref, b2_ref, y_ref):
    del blk_e_ref
    h = jnp.dot(xg_ref[...], w1_ref[0], preferred_element_type=jnp.float32)
    h = jnp.maximum(h + b1_ref[0], 0.0)
    y_ref[...] = (jnp.dot(h, w2_ref[0], preferred_element_type=jnp.float32)
                  + b2_ref[0])


def _dispatch_kernel(x_hbm, pos_hbm, xg_hbm, p0_v, p1_v, xbuf, gsem, wsem):
    wid = lax.axis_index("s") * 2 + lax.axis_index("c")
    base = wid * TPW
    pltpu.sync_copy(pos_hbm.at[wid, 0], p0_v)
    pltpu.sync_copy(pos_hbm.at[wid, 1], p1_v)
    pltpu.async_copy(x_hbm.at[pl.ds(base, TPW)], xbuf, gsem).wait()
    s0 = pltpu.async_copy(xbuf, xg_hbm.at[p0_v], wsem)
    s1 = pltpu.async_copy(xbuf, xg_hbm.at[p1_v], wsem)
    s0.wait()
    s1.wait()


def _combine_kernel(y_hbm, pos_hbm, pw_hbm, out_hbm, p0_v, p1_v, w0_v, w1_v,
                    buf0, buf1, sem):
    wid = lax.axis_index("s") * 2 + lax.axis_index("c")
    base = wid * TPW
    pltpu.sync_copy(pos_hbm.at[wid, 0], p0_v)
    pltpu.sync_copy(pos_hbm.at[wid, 1], p1_v)
    pltpu.sync_copy(pw_hbm.at[wid, 0], w0_v)
    pltpu.sync_copy(pw_hbm.at[wid, 1], w1_v)
    c0 = pltpu.async_copy(y_hbm.at[p0_v], buf0, sem)
    c1 = pltpu.async_copy(y_hbm.at[p1_v], buf1, sem)
    c0.wait()
    c1.wait()

    def row(r, carry):
        w0 = w0_v[pl.ds(r * 16, 16)]
        w1 = w1_v[pl.ds(r * 16, 16)]
        for j in range(EMBED_DIM // 16):
            sl = pl.ds(j * 16, 16)
            buf0[r, sl] = w0 * buf0[r, sl] + w1 * buf1[r, sl]
        return carry

    lax.fori_loop(0, TPW, row, 0)
    pltpu.sync_copy(buf0, out_hbm.at[pl.ds(base, TPW)])


def kernel(x, Wr, br, W1, b1, W2, b2):
    batch, seq, d = x.shape
    x_flat = x.reshape(-1, d)
    T = x_flat.shape[0]

    eidx, pw = pl.pallas_call(
        _router_kernel,
        grid=(1,),
        in_specs=[
            pl.BlockSpec((T, d), lambda i: (0, 0)),
            pl.BlockSpec((d, NUM_EXPERTS), lambda i: (0, 0)),
            pl.BlockSpec((1, NUM_EXPERTS), lambda i: (0, 0)),
        ],
        out_specs=[
            pl.BlockSpec((T, TOP_K), lambda i: (0, 0)),
            pl.BlockSpec((T, TOP_K), lambda i: (0, 0)),
        ],
        out_shape=[
            jax.ShapeDtypeStruct((T, TOP_K), jnp.int32),
            jax.ShapeDtypeStruct((T, TOP_K), jnp.float32),
        ],
    )(x_flat, Wr, br.reshape(1, NUM_EXPERTS))

    blk_e, pos = _plan(eidx)
    pwT = pw.reshape(NW, TPW, TOP_K).transpose(0, 2, 1)
    pwb = jnp.broadcast_to(pwT[..., None],
                           (NW, TOP_K, TPW, 16)).reshape(NW, TOP_K, TPW * 16)

    mesh = plsc.VectorSubcoreMesh(core_axis_name="c", subcore_axis_name="s")

    xg = pl.kernel(
        _dispatch_kernel,
        mesh=mesh,
        out_type=jax.ShapeDtypeStruct((NP_ROWS, d), jnp.float32),
        scratch_types=[
            pltpu.VMEM((TPW,), jnp.int32),
            pltpu.VMEM((TPW,), jnp.int32),
            pltpu.VMEM((TPW, d), jnp.float32),
            pltpu.SemaphoreType.DMA,
            pltpu.SemaphoreType.DMA,
        ],
    )(x_flat, pos)

    y = pl.pallas_call(
        _ffn_kernel,
        grid_spec=pltpu.PrefetchScalarGridSpec(
            num_scalar_prefetch=1,
            grid=(NG,),
            in_specs=[
                pl.BlockSpec((BT, d), lambda g, be: (g, 0)),
                pl.BlockSpec((1, d, HIDDEN_DIM), lambda g, be: (be[g], 0, 0)),
                pl.BlockSpec((1, 1, HIDDEN_DIM), lambda g, be: (be[g], 0, 0)),
                pl.BlockSpec((1, HIDDEN_DIM, d), lambda g, be: (be[g], 0, 0)),
                pl.BlockSpec((1, 1, d), lambda g, be: (be[g], 0, 0)),
            ],
            out_specs=pl.BlockSpec((BT, d), lambda g, be: (g, 0)),
        ),
        out_shape=jax.ShapeDtypeStruct((NP_ROWS, d), jnp.float32),
    )(blk_e, xg, W1,
      b1.reshape(NUM_EXPERTS, 1, HIDDEN_DIM), W2,
      b2.reshape(NUM_EXPERTS, 1, EMBED_DIM))

    out = pl.kernel(
        _combine_kernel,
        mesh=mesh,
        out_type=jax.ShapeDtypeStruct((T, d), jnp.float32),
        scratch_types=[
            pltpu.VMEM((TPW,), jnp.int32),
            pltpu.VMEM((TPW,), jnp.int32),
            pltpu.VMEM((TPW * 16,), jnp.float32),
            pltpu.VMEM((TPW * 16,), jnp.float32),
            pltpu.VMEM((TPW, d), jnp.float32),
            pltpu.VMEM((TPW, d), jnp.float32),
            pltpu.SemaphoreType.DMA,
        ],
    )(y, pos, pwb)

    return out.reshape(batch, seq, d)


def _plan(eidx):
    e_flat = eidx.reshape(-1)
    oh = (e_flat[:, None] == jnp.arange(NUM_EXPERTS)[None, :]).astype(jnp.int32)
    cnt_incl = jnp.cumsum(oh, axis=0)
    cnt_before = jnp.take_along_axis(cnt_incl, e_flat[:, None], 1)[:, 0] - 1
    counts = cnt_incl[-1]
    nb = (counts + BT - 1) // BT
    bounds = jnp.concatenate([jnp.zeros((1,), jnp.int32),
                              jnp.cumsum(nb)[:-1]]).astype(jnp.int32)
    pad_base = BT * bounds
    padpos = (jnp.sum(oh * pad_base[None, :], axis=1) + cnt_before)
    blk_e = jnp.clip(
        jnp.sum(jnp.arange(NG, dtype=jnp.int32)[:, None] >= bounds[None, :],
                axis=1) - 1,
        0, NUM_EXPERTS - 1).astype(jnp.int32)
    pos = padpos.reshape(NW, TPW, TOP_K).transpose(0, 2, 1)
    return blk_e, pos


# speedup vs baseline: 1.6748x; 1.0009x over previous
import functools

import jax
import jax.numpy as jnp
from jax import lax
from jax.experimental import pallas as pl
from jax.experimental.pallas import tpu as pltpu
from jax.experimental.pallas import tpu_sc as plsc

EMBED_DIM = 768
HIDDEN_DIM = 3072
NUM_EXPERTS = 8
TOP_K = 2

BT = 128
T_TOKENS = 2048
NPAIR = T_TOKENS * TOP_K
NG = NPAIR // BT + NUM_EXPERTS
NP_ROWS = NG * BT

NW = 32
TPW = T_TOKENS // NW


def _router_kernel(x_ref, wr_ref, br_ref, eidx_ref, pw_ref):
    logits = jnp.dot(x_ref[...], wr_ref[...],
                     preferred_element_type=jnp.float32) + br_ref[...]
    lane = lax.broadcasted_iota(jnp.int32, logits.shape, 1)
    m1 = jnp.max(logits, axis=1, keepdims=True)
    i1 = jnp.min(jnp.where(logits == m1, lane, NUM_EXPERTS), axis=1,
                 keepdims=True)
    l2 = jnp.where(lane == i1, -jnp.inf, logits)
    m2 = jnp.max(l2, axis=1, keepdims=True)
    i2 = jnp.min(jnp.where(l2 == m2, lane, NUM_EXPERTS), axis=1, keepdims=True)
    p2 = 1.0 / (1.0 + jnp.exp(m1 - m2))
    p1 = 1.0 - p2
    eidx_ref[...] = jnp.concatenate([i1, i2], axis=1)
    pw_ref[...] = jnp.concatenate([p1, p2], axis=1)


def _ffn_kernel(blk_e_ref, xg_ref, w1_ref, b1_ref, w2_ref, b2_ref, y_ref):
    del blk_e_ref
    h = jnp.dot(xg_ref[...].astype(jnp.bfloat16),
                w1_ref[0].astype(jnp.bfloat16),
                preferred_element_type=jnp.float32)
    h = jnp.maximum(h + b1_ref[0], 0.0)
    y_ref[...] = (jnp.dot(h.astype(jnp.bfloat16),
                          w2_ref[0].astype(jnp.bfloat16),
                          preferred_element_type=jnp.float32)
                  + b2_ref[0])


def _dispatch_kernel(x_hbm, pos_hbm, xg_hbm, p0_v, p1_v, xbuf, gsem, wsem):
    wid = lax.axis_index("s") * 2 + lax.axis_index("c")
    base = wid * TPW
    pltpu.sync_copy(pos_hbm.at[wid, 0], p0_v)
    pltpu.sync_copy(pos_hbm.at[wid, 1], p1_v)
    pltpu.async_copy(x_hbm.at[pl.ds(base, TPW)], xbuf, gsem).wait()
    s0 = pltpu.async_copy(xbuf, xg_hbm.at[p0_v], wsem)
    s1 = pltpu.async_copy(xbuf, xg_hbm.at[p1_v], wsem)
    s0.wait()
    s1.wait()


def _combine_kernel(y_hbm, pos_hbm, pw_hbm, out_hbm, p0_v, p1_v, w0_v, w1_v,
                    buf0, buf1, sem):
    wid = lax.axis_index("s") * 2 + lax.axis_index("c")
    base = wid * TPW
    pltpu.sync_copy(pos_hbm.at[wid, 0], p0_v)
    pltpu.sync_copy(pos_hbm.at[wid, 1], p1_v)
    pltpu.sync_copy(pw_hbm.at[wid, 0], w0_v)
    pltpu.sync_copy(pw_hbm.at[wid, 1], w1_v)
    c0 = pltpu.async_copy(y_hbm.at[p0_v], buf0, sem)
    c1 = pltpu.async_copy(y_hbm.at[p1_v], buf1, sem)
    c0.wait()
    c1.wait()

    def row(r, carry):
        w0 = w0_v[pl.ds(r * 16, 16)]
        w1 = w1_v[pl.ds(r * 16, 16)]
        for j in range(EMBED_DIM // 16):
            sl = pl.ds(j * 16, 16)
            buf0[r, sl] = w0 * buf0[r, sl] + w1 * buf1[r, sl]
        return carry

    lax.fori_loop(0, TPW, row, 0)
    pltpu.sync_copy(buf0, out_hbm.at[pl.ds(base, TPW)])


def kernel(x, Wr, br, W1, b1, W2, b2):
    batch, seq, d = x.shape
    x_flat = x.reshape(-1, d)
    T = x_flat.shape[0]

    eidx, pw = pl.pallas_call(
        _router_kernel,
        grid=(1,),
        in_specs=[
            pl.BlockSpec((T, d), lambda i: (0, 0)),
            pl.BlockSpec((d, NUM_EXPERTS), lambda i: (0, 0)),
            pl.BlockSpec((1, NUM_EXPERTS), lambda i: (0, 0)),
        ],
        out_specs=[
            pl.BlockSpec((T, TOP_K), lambda i: (0, 0)),
            pl.BlockSpec((T, TOP_K), lambda i: (0, 0)),
        ],
        out_shape=[
            jax.ShapeDtypeStruct((T, TOP_K), jnp.int32),
            jax.ShapeDtypeStruct((T, TOP_K), jnp.float32),
        ],
    )(x_flat, Wr, br.reshape(1, NUM_EXPERTS))

    blk_e, pos = _plan(eidx)
    pwT = pw.reshape(NW, TPW, TOP_K).transpose(0, 2, 1)
    pwb = jnp.broadcast_to(pwT[..., None],
                           (NW, TOP_K, TPW, 16)).reshape(NW, TOP_K, TPW * 16)

    mesh = plsc.VectorSubcoreMesh(core_axis_name="c", subcore_axis_name="s")

    xg = pl.kernel(
        _dispatch_kernel,
        mesh=mesh,
        out_type=jax.ShapeDtypeStruct((NP_ROWS, d), jnp.float32),
        scratch_types=[
            pltpu.VMEM((TPW,), jnp.int32),
            pltpu.VMEM((TPW,), jnp.int32),
            pltpu.VMEM((TPW, d), jnp.float32),
            pltpu.SemaphoreType.DMA,
            pltpu.SemaphoreType.DMA,
        ],
    )(x_flat, pos)

    y = pl.pallas_call(
        _ffn_kernel,
        grid_spec=pltpu.PrefetchScalarGridSpec(
            num_scalar_prefetch=1,
            grid=(NG,),
            in_specs=[
                pl.BlockSpec((BT, d), lambda g, be: (g, 0)),
                pl.BlockSpec((1, d, HIDDEN_DIM), lambda g, be: (be[g], 0, 0)),
                pl.BlockSpec((1, 1, HIDDEN_DIM), lambda g, be: (be[g], 0, 0)),
                pl.BlockSpec((1, HIDDEN_DIM, d), lambda g, be: (be[g], 0, 0)),
                pl.BlockSpec((1, 1, d), lambda g, be: (be[g], 0, 0)),
            ],
            out_specs=pl.BlockSpec((BT, d), lambda g, be: (g, 0)),
        ),
        out_shape=jax.ShapeDtypeStruct((NP_ROWS, d), jnp.float32),
    )(blk_e, xg, W1,
      b1.reshape(NUM_EXPERTS, 1, HIDDEN_DIM), W2,
      b2.reshape(NUM_EXPERTS, 1, EMBED_DIM))

    out = pl.kernel(
        _combine_kernel,
        mesh=mesh,
        out_type=jax.ShapeDtypeStruct((T, d), jnp.float32),
        scratch_types=[
            pltpu.VMEM((TPW,), jnp.int32),
            pltpu.VMEM((TPW,), jnp.int32),
            pltpu.VMEM((TPW * 16,), jnp.float32),
            pltpu.VMEM((TPW * 16,), jnp.float32),
            pltpu.VMEM((TPW, d), jnp.float32),
            pltpu.VMEM((TPW, d), jnp.float32),
            pltpu.SemaphoreType.DMA,
        ],
    )(y, pos, pwb)

    return out.reshape(batch, seq, d)


def _plan(eidx):
    e_flat = eidx.reshape(-1)
    oh = (e_flat[:, None] == jnp.arange(NUM_EXPERTS)[None, :]).astype(jnp.int32)
    cnt_incl = jnp.cumsum(oh, axis=0)
    cnt_before = jnp.take_along_axis(cnt_incl, e_flat[:, None], 1)[:, 0] - 1
    counts = cnt_incl[-1]
    nb = (counts + BT - 1) // BT
    bounds = jnp.concatenate([jnp.zeros((1,), jnp.int32),
                              jnp.cumsum(nb)[:-1]]).astype(jnp.int32)
    pad_base = BT * bounds
    padpos = (jnp.sum(oh * pad_base[None, :], axis=1) + cnt_before)
    blk_e = jnp.clip(
        jnp.sum(jnp.arange(NG, dtype=jnp.int32)[:, None] >= bounds[None, :],
                axis=1) - 1,
        0, NUM_EXPERTS - 1).astype(jnp.int32)
    pos = padpos.reshape(NW, TPW, TOP_K).transpose(0, 2, 1)
    return blk_e, pos


# speedup vs baseline: 1.7736x; 1.0589x over previous
import functools

import jax
import jax.numpy as jnp
from jax import lax
from jax.experimental import pallas as pl
from jax.experimental.pallas import tpu as pltpu
from jax.experimental.pallas import tpu_sc as plsc

EMBED_DIM = 768
HIDDEN_DIM = 3072
NUM_EXPERTS = 8
TOP_K = 2

BT = 128
T_TOKENS = 2048
NPAIR = T_TOKENS * TOP_K
NG = NPAIR // BT + NUM_EXPERTS
NP_ROWS = NG * BT

NW = 32
TPW = T_TOKENS // NW

CB = 128
NCB = T_TOKENS // CB


def _router_kernel(x_ref, wr_ref, br_ref,
                   pos0_ref, pos1_ref, pw0_ref, pw1_ref, blk_ref):
    T = T_TOKENS
    logits = jnp.dot(x_ref[...], wr_ref[...],
                     preferred_element_type=jnp.float32) + br_ref[...]
    lane = lax.broadcasted_iota(jnp.int32, logits.shape, 1)
    m1 = jnp.max(logits, axis=1, keepdims=True)
    i1 = jnp.min(jnp.where(logits == m1, lane, NUM_EXPERTS), axis=1,
                 keepdims=True)
    l2 = jnp.where(lane == i1, -jnp.inf, logits)
    m2 = jnp.max(l2, axis=1, keepdims=True)
    i2 = jnp.min(jnp.where(l2 == m2, lane, NUM_EXPERTS), axis=1, keepdims=True)
    p2 = 1.0 / (1.0 + jnp.exp(m1 - m2))
    p1 = 1.0 - p2

    oh1 = jnp.where(lane == i1, 1.0, 0.0)
    oh2 = jnp.where(lane == i2, 1.0, 0.0)
    m_both = oh1 + oh2

    r_i = lax.broadcasted_iota(jnp.int32, (CB, CB), 0)
    c_i = lax.broadcasted_iota(jnp.int32, (CB, CB), 1)
    tri = jnp.where(r_i > c_i, 1.0, 0.0)
    tot = jnp.zeros((1, NUM_EXPERTS), jnp.float32)
    parts = []
    for c in range(NCB):
        mc = m_both[c * CB:(c + 1) * CB]
        parts.append(jnp.dot(tri, mc, preferred_element_type=jnp.float32)
                     + tot)
        tot = tot + jnp.sum(mc, axis=0, keepdims=True)
    s_excl = jnp.concatenate(parts, axis=0)
    counts = tot

    nb = jnp.floor((counts + (BT - 1)) * (1.0 / BT))
    r8 = lax.broadcasted_iota(jnp.int32, (NUM_EXPERTS, NUM_EXPERTS), 0)
    c8 = lax.broadcasted_iota(jnp.int32, (NUM_EXPERTS, NUM_EXPERTS), 1)
    lt8 = jnp.where(r8 < c8, 1.0, 0.0)
    bounds = jnp.dot(nb, lt8, preferred_element_type=jnp.float32)
    pad_base = bounds * float(BT)

    pos0 = (jnp.sum(oh1 * pad_base, axis=1, keepdims=True)
            + jnp.sum(oh1 * s_excl, axis=1, keepdims=True))
    pos1 = (jnp.sum(oh2 * pad_base, axis=1, keepdims=True)
            + jnp.sum(oh2 * s_excl, axis=1, keepdims=True))
    pos0_ref[...] = pos0.astype(jnp.int32)
    pos1_ref[...] = pos1.astype(jnp.int32)
    pw0_ref[...] = jnp.broadcast_to(p1, (T, 16))
    pw1_ref[...] = jnp.broadcast_to(p2, (T, 16))

    gi = lax.broadcasted_iota(jnp.int32, (NG, NUM_EXPERTS), 0)
    ge = jnp.where(gi >= bounds.astype(jnp.int32), 1.0, 0.0)
    blk = jnp.sum(ge, axis=1, keepdims=True) - 1.0
    blk_ref[...] = jnp.clip(blk, 0.0, NUM_EXPERTS - 1).astype(jnp.int32)


def _ffn_kernel(blk_e_ref, xg_ref, w1_ref, b1_ref, w2_ref, b2_ref, y_ref):
    del blk_e_ref
    h = jnp.dot(xg_ref[...], w1_ref[0], preferred_element_type=jnp.float32)
    h = jnp.maximum(h + b1_ref[0], 0.0)
    y_ref[...] = (jnp.dot(h, w2_ref[0], preferred_element_type=jnp.float32)
                  + b2_ref[0])


def _dispatch_kernel(x_hbm, p0_hbm, p1_hbm, xg_hbm, p0_v, p1_v, xbuf,
                     gsem, wsem):
    wid = lax.axis_index("s") * 2 + lax.axis_index("c")
    base = wid * TPW
    pltpu.sync_copy(p0_hbm.at[pl.ds(base, TPW)], p0_v)
    pltpu.sync_copy(p1_hbm.at[pl.ds(base, TPW)], p1_v)
    pltpu.async_copy(x_hbm.at[pl.ds(base, TPW)], xbuf, gsem).wait()
    s0 = pltpu.async_copy(xbuf, xg_hbm.at[p0_v], wsem)
    s1 = pltpu.async_copy(xbuf, xg_hbm.at[p1_v], wsem)
    s0.wait()
    s1.wait()


def _combine_kernel(y_hbm, p0_hbm, p1_hbm, pw0_hbm, pw1_hbm, out_hbm,
                    p0_v, p1_v, w0_v, w1_v, buf0, buf1, sem):
    wid = lax.axis_index("s") * 2 + lax.axis_index("c")
    base = wid * TPW
    pltpu.sync_copy(p0_hbm.at[pl.ds(base, TPW)], p0_v)
    pltpu.sync_copy(p1_hbm.at[pl.ds(base, TPW)], p1_v)
    pltpu.sync_copy(pw0_hbm.at[pl.ds(base * 16, TPW * 16)], w0_v)
    pltpu.sync_copy(pw1_hbm.at[pl.ds(base * 16, TPW * 16)], w1_v)
    c0 = pltpu.async_copy(y_hbm.at[p0_v], buf0, sem)
    c1 = pltpu.async_copy(y_hbm.at[p1_v], buf1, sem)
    c0.wait()
    c1.wait()

    def row(r, carry):
        w0 = w0_v[pl.ds(r * 16, 16)]
        w1 = w1_v[pl.ds(r * 16, 16)]
        for j in range(EMBED_DIM // 16):
            sl = pl.ds(j * 16, 16)
            buf0[r, sl] = w0 * buf0[r, sl] + w1 * buf1[r, sl]
        return carry

    lax.fori_loop(0, TPW, row, 0)
    pltpu.sync_copy(buf0, out_hbm.at[pl.ds(base, TPW)])


def kernel(x, Wr, br, W1, b1, W2, b2):
    batch, seq, d = x.shape
    x_flat = x.reshape(-1, d)
    T = x_flat.shape[0]

    pos0, pos1, pw0b, pw1b, blk_e = pl.pallas_call(
        _router_kernel,
        grid=(1,),
        in_specs=[
            pl.BlockSpec((T, d), lambda i: (0, 0)),
            pl.BlockSpec((d, NUM_EXPERTS), lambda i: (0, 0)),
            pl.BlockSpec((1, NUM_EXPERTS), lambda i: (0, 0)),
        ],
        out_specs=[
            pl.BlockSpec((T, 1), lambda i: (0, 0)),
            pl.BlockSpec((T, 1), lambda i: (0, 0)),
            pl.BlockSpec((T, 16), lambda i: (0, 0)),
            pl.BlockSpec((T, 16), lambda i: (0, 0)),
            pl.BlockSpec((NG, 1), lambda i: (0, 0)),
        ],
        out_shape=[
            jax.ShapeDtypeStruct((T, 1), jnp.int32),
            jax.ShapeDtypeStruct((T, 1), jnp.int32),
            jax.ShapeDtypeStruct((T, 16), jnp.float32),
            jax.ShapeDtypeStruct((T, 16), jnp.float32),
            jax.ShapeDtypeStruct((NG, 1), jnp.int32),
        ],
    )(x_flat, Wr, br.reshape(1, NUM_EXPERTS))

    pos0 = pos0.reshape(T)
    pos1 = pos1.reshape(T)
    pw0b = pw0b.reshape(T * 16)
    pw1b = pw1b.reshape(T * 16)
    blk_e = blk_e.reshape(NG)

    mesh = plsc.VectorSubcoreMesh(core_axis_name="c", subcore_axis_name="s")

    xg = pl.kernel(
        _dispatch_kernel,
        mesh=mesh,
        out_type=jax.ShapeDtypeStruct((NP_ROWS, d), jnp.float32),
        scratch_types=[
            pltpu.VMEM((TPW,), jnp.int32),
            pltpu.VMEM((TPW,), jnp.int32),
            pltpu.VMEM((TPW, d), jnp.float32),
            pltpu.SemaphoreType.DMA,
            pltpu.SemaphoreType.DMA,
        ],
    )(x_flat, pos0, pos1)

    y = pl.pallas_call(
        _ffn_kernel,
        grid_spec=pltpu.PrefetchScalarGridSpec(
            num_scalar_prefetch=1,
            grid=(NG,),
            in_specs=[
                pl.BlockSpec((BT, d), lambda g, be: (g, 0)),
                pl.BlockSpec((1, d, HIDDEN_DIM), lambda g, be: (be[g], 0, 0)),
                pl.BlockSpec((1, 1, HIDDEN_DIM), lambda g, be: (be[g], 0, 0)),
                pl.BlockSpec((1, HIDDEN_DIM, d), lambda g, be: (be[g], 0, 0)),
                pl.BlockSpec((1, 1, d), lambda g, be: (be[g], 0, 0)),
            ],
            out_specs=pl.BlockSpec((BT, d), lambda g, be: (g, 0)),
        ),
        out_shape=jax.ShapeDtypeStruct((NP_ROWS, d), jnp.float32),
    )(blk_e, xg, W1,
      b1.reshape(NUM_EXPERTS, 1, HIDDEN_DIM), W2,
      b2.reshape(NUM_EXPERTS, 1, EMBED_DIM))

    out = pl.kernel(
        _combine_kernel,
        mesh=mesh,
        out_type=jax.ShapeDtypeStruct((T, d), jnp.float32),
        scratch_types=[
            pltpu.VMEM((TPW,), jnp.int32),
            pltpu.VMEM((TPW,), jnp.int32),
            pltpu.VMEM((TPW * 16,), jnp.float32),
            pltpu.VMEM((TPW * 16,), jnp.float32),
            pltpu.VMEM((TPW, d), jnp.float32),
            pltpu.VMEM((TPW, d), jnp.float32),
            pltpu.SemaphoreType.DMA,
        ],
    )(y, pos0, pos1, pw0b, pw1b)

    return out.reshape(batch, seq, d)


# speedup vs baseline: 1.8451x; 1.0404x over previous
import functools

import jax
import jax.numpy as jnp
from jax import lax
from jax.experimental import pallas as pl
from jax.experimental.pallas import tpu as pltpu
from jax.experimental.pallas import tpu_sc as plsc

EMBED_DIM = 768
HIDDEN_DIM = 3072
NUM_EXPERTS = 8
TOP_K = 2

BT = 256
T_TOKENS = 2048
NPAIR = T_TOKENS * TOP_K
NG = NPAIR // BT + NUM_EXPERTS
NP_ROWS = NG * BT

NW = 32
TPW = T_TOKENS // NW

CB = 128
NCB = T_TOKENS // CB


def _router_kernel(x_ref, wr_ref, br_ref,
                   pos0_ref, pos1_ref, pw0_ref, pw1_ref, blk_ref):
    T = T_TOKENS
    logits = jnp.dot(x_ref[...], wr_ref[...],
                     preferred_element_type=jnp.float32) + br_ref[...]
    lane = lax.broadcasted_iota(jnp.int32, logits.shape, 1)
    m1 = jnp.max(logits, axis=1, keepdims=True)
    i1 = jnp.min(jnp.where(logits == m1, lane, NUM_EXPERTS), axis=1,
                 keepdims=True)
    l2 = jnp.where(lane == i1, -jnp.inf, logits)
    m2 = jnp.max(l2, axis=1, keepdims=True)
    i2 = jnp.min(jnp.where(l2 == m2, lane, NUM_EXPERTS), axis=1, keepdims=True)
    p2 = 1.0 / (1.0 + jnp.exp(m1 - m2))
    p1 = 1.0 - p2

    oh1 = jnp.where(lane == i1, 1.0, 0.0)
    oh2 = jnp.where(lane == i2, 1.0, 0.0)
    m_both = oh1 + oh2

    r_i = lax.broadcasted_iota(jnp.int32, (CB, CB), 0)
    c_i = lax.broadcasted_iota(jnp.int32, (CB, CB), 1)
    tri = jnp.where(r_i > c_i, 1.0, 0.0)
    tot = jnp.zeros((1, NUM_EXPERTS), jnp.float32)
    parts = []
    for c in range(NCB):
        mc = m_both[c * CB:(c + 1) * CB]
        parts.append(jnp.dot(tri, mc, preferred_element_type=jnp.float32)
                     + tot)
        tot = tot + jnp.sum(mc, axis=0, keepdims=True)
    s_excl = jnp.concatenate(parts, axis=0)
    counts = tot

    nb = jnp.floor((counts + (BT - 1)) * (1.0 / BT))
    r8 = lax.broadcasted_iota(jnp.int32, (NUM_EXPERTS, NUM_EXPERTS), 0)
    c8 = lax.broadcasted_iota(jnp.int32, (NUM_EXPERTS, NUM_EXPERTS), 1)
    lt8 = jnp.where(r8 < c8, 1.0, 0.0)
    bounds = jnp.dot(nb, lt8, preferred_element_type=jnp.float32)
    pad_base = bounds * float(BT)

    pos0 = (jnp.sum(oh1 * pad_base, axis=1, keepdims=True)
            + jnp.sum(oh1 * s_excl, axis=1, keepdims=True))
    pos1 = (jnp.sum(oh2 * pad_base, axis=1, keepdims=True)
            + jnp.sum(oh2 * s_excl, axis=1, keepdims=True))
    pos0_ref[...] = pos0.astype(jnp.int32)
    pos1_ref[...] = pos1.astype(jnp.int32)
    pw0_ref[...] = jnp.broadcast_to(p1, (T, 16))
    pw1_ref[...] = jnp.broadcast_to(p2, (T, 16))

    gi = lax.broadcasted_iota(jnp.int32, (NG, NUM_EXPERTS), 0)
    ge = jnp.where(gi >= bounds.astype(jnp.int32), 1.0, 0.0)
    blk = jnp.sum(ge, axis=1, keepdims=True) - 1.0
    blk_ref[...] = jnp.clip(blk, 0.0, NUM_EXPERTS - 1).astype(jnp.int32)


def _ffn_kernel(blk_e_ref, xg_ref, w1_ref, b1_ref, w2_ref, b2_ref, y_ref):
    del blk_e_ref
    h = jnp.dot(xg_ref[...], w1_ref[0], preferred_element_type=jnp.float32)
    h = jnp.maximum(h + b1_ref[0], 0.0)
    y_ref[...] = (jnp.dot(h, w2_ref[0], preferred_element_type=jnp.float32)
                  + b2_ref[0])


def _dispatch_kernel(x_hbm, p0_hbm, p1_hbm, xg_hbm, p0_v, p1_v, xbuf,
                     gsem, wsem):
    wid = lax.axis_index("s") * 2 + lax.axis_index("c")
    base = wid * TPW
    pltpu.sync_copy(p0_hbm.at[pl.ds(base, TPW)], p0_v)
    pltpu.sync_copy(p1_hbm.at[pl.ds(base, TPW)], p1_v)
    pltpu.async_copy(x_hbm.at[pl.ds(base, TPW)], xbuf, gsem).wait()
    s0 = pltpu.async_copy(xbuf, xg_hbm.at[p0_v], wsem)
    s1 = pltpu.async_copy(xbuf, xg_hbm.at[p1_v], wsem)
    s0.wait()
    s1.wait()


def _combine_kernel(y_hbm, p0_hbm, p1_hbm, pw0_hbm, pw1_hbm, out_hbm,
                    p0_v, p1_v, w0_v, w1_v, buf0, buf1, sem):
    wid = lax.axis_index("s") * 2 + lax.axis_index("c")
    base = wid * TPW
    pltpu.sync_copy(p0_hbm.at[pl.ds(base, TPW)], p0_v)
    pltpu.sync_copy(p1_hbm.at[pl.ds(base, TPW)], p1_v)
    pltpu.sync_copy(pw0_hbm.at[pl.ds(base * 16, TPW * 16)], w0_v)
    pltpu.sync_copy(pw1_hbm.at[pl.ds(base * 16, TPW * 16)], w1_v)
    c0 = pltpu.async_copy(y_hbm.at[p0_v], buf0, sem)
    c1 = pltpu.async_copy(y_hbm.at[p1_v], buf1, sem)
    c0.wait()
    c1.wait()

    def row(r, carry):
        w0 = w0_v[pl.ds(r * 16, 16)]
        w1 = w1_v[pl.ds(r * 16, 16)]
        for j in range(EMBED_DIM // 16):
            sl = pl.ds(j * 16, 16)
            buf0[r, sl] = w0 * buf0[r, sl] + w1 * buf1[r, sl]
        return carry

    lax.fori_loop(0, TPW, row, 0)
    pltpu.sync_copy(buf0, out_hbm.at[pl.ds(base, TPW)])


def kernel(x, Wr, br, W1, b1, W2, b2):
    batch, seq, d = x.shape
    x_flat = x.reshape(-1, d)
    T = x_flat.shape[0]

    pos0, pos1, pw0b, pw1b, blk_e = pl.pallas_call(
        _router_kernel,
        grid=(1,),
        in_specs=[
            pl.BlockSpec((T, d), lambda i: (0, 0)),
            pl.BlockSpec((d, NUM_EXPERTS), lambda i: (0, 0)),
            pl.BlockSpec((1, NUM_EXPERTS), lambda i: (0, 0)),
        ],
        out_specs=[
            pl.BlockSpec((T, 1), lambda i: (0, 0)),
            pl.BlockSpec((T, 1), lambda i: (0, 0)),
            pl.BlockSpec((T, 16), lambda i: (0, 0)),
            pl.BlockSpec((T, 16), lambda i: (0, 0)),
            pl.BlockSpec((NG, 1), lambda i: (0, 0)),
        ],
        out_shape=[
            jax.ShapeDtypeStruct((T, 1), jnp.int32),
            jax.ShapeDtypeStruct((T, 1), jnp.int32),
            jax.ShapeDtypeStruct((T, 16), jnp.float32),
            jax.ShapeDtypeStruct((T, 16), jnp.float32),
            jax.ShapeDtypeStruct((NG, 1), jnp.int32),
        ],
    )(x_flat, Wr, br.reshape(1, NUM_EXPERTS))

    pos0 = pos0.reshape(T)
    pos1 = pos1.reshape(T)
    pw0b = pw0b.reshape(T * 16)
    pw1b = pw1b.reshape(T * 16)
    blk_e = blk_e.reshape(NG)

    mesh = plsc.VectorSubcoreMesh(core_axis_name="c", subcore_axis_name="s")

    xg = pl.kernel(
        _dispatch_kernel,
        mesh=mesh,
        out_type=jax.ShapeDtypeStruct((NP_ROWS, d), jnp.float32),
        scratch_types=[
            pltpu.VMEM((TPW,), jnp.int32),
            pltpu.VMEM((TPW,), jnp.int32),
            pltpu.VMEM((TPW, d), jnp.float32),
            pltpu.SemaphoreType.DMA,
            pltpu.SemaphoreType.DMA,
        ],
    )(x_flat, pos0, pos1)

    y = pl.pallas_call(
        _ffn_kernel,
        grid_spec=pltpu.PrefetchScalarGridSpec(
            num_scalar_prefetch=1,
            grid=(NG,),
            in_specs=[
                pl.BlockSpec((BT, d), lambda g, be: (g, 0)),
                pl.BlockSpec((1, d, HIDDEN_DIM), lambda g, be: (be[g], 0, 0)),
                pl.BlockSpec((1, 1, HIDDEN_DIM), lambda g, be: (be[g], 0, 0)),
                pl.BlockSpec((1, HIDDEN_DIM, d), lambda g, be: (be[g], 0, 0)),
                pl.BlockSpec((1, 1, d), lambda g, be: (be[g], 0, 0)),
            ],
            out_specs=pl.BlockSpec((BT, d), lambda g, be: (g, 0)),
        ),
        out_shape=jax.ShapeDtypeStruct((NP_ROWS, d), jnp.float32),
    )(blk_e, xg, W1,
      b1.reshape(NUM_EXPERTS, 1, HIDDEN_DIM), W2,
      b2.reshape(NUM_EXPERTS, 1, EMBED_DIM))

    out = pl.kernel(
        _combine_kernel,
        mesh=mesh,
        out_type=jax.ShapeDtypeStruct((T, d), jnp.float32),
        scratch_types=[
            pltpu.VMEM((TPW,), jnp.int32),
            pltpu.VMEM((TPW,), jnp.int32),
            pltpu.VMEM((TPW * 16,), jnp.float32),
            pltpu.VMEM((TPW * 16,), jnp.float32),
            pltpu.VMEM((TPW, d), jnp.float32),
            pltpu.VMEM((TPW, d), jnp.float32),
            pltpu.SemaphoreType.DMA,
        ],
    )(y, pos0, pos1, pw0b, pw1b)

    return out.reshape(batch, seq, d)


# speedup vs baseline: 1.8951x; 1.0271x over previous
import functools

import jax
import jax.numpy as jnp
from jax import lax
from jax.experimental import pallas as pl
from jax.experimental.pallas import tpu as pltpu
from jax.experimental.pallas import tpu_sc as plsc

EMBED_DIM = 768
HIDDEN_DIM = 3072
NUM_EXPERTS = 8
TOP_K = 2

BT = 512
T_TOKENS = 2048
NPAIR = T_TOKENS * TOP_K
NG = NPAIR // BT + NUM_EXPERTS
NP_ROWS = NG * BT

NW = 32
TPW = T_TOKENS // NW

CB = 128
NCB = T_TOKENS // CB


def _router_kernel(x_ref, wr_ref, br_ref,
                   pos0_ref, pos1_ref, pw0_ref, pw1_ref, blk_ref):
    T = T_TOKENS
    logits = jnp.dot(x_ref[...], wr_ref[...],
                     preferred_element_type=jnp.float32) + br_ref[...]
    lane = lax.broadcasted_iota(jnp.int32, logits.shape, 1)
    m1 = jnp.max(logits, axis=1, keepdims=True)
    i1 = jnp.min(jnp.where(logits == m1, lane, NUM_EXPERTS), axis=1,
                 keepdims=True)
    l2 = jnp.where(lane == i1, -jnp.inf, logits)
    m2 = jnp.max(l2, axis=1, keepdims=True)
    i2 = jnp.min(jnp.where(l2 == m2, lane, NUM_EXPERTS), axis=1, keepdims=True)
    p2 = 1.0 / (1.0 + jnp.exp(m1 - m2))
    p1 = 1.0 - p2

    oh1 = jnp.where(lane == i1, 1.0, 0.0)
    oh2 = jnp.where(lane == i2, 1.0, 0.0)
    m_both = oh1 + oh2

    r_i = lax.broadcasted_iota(jnp.int32, (CB, CB), 0)
    c_i = lax.broadcasted_iota(jnp.int32, (CB, CB), 1)
    tri = jnp.where(r_i > c_i, 1.0, 0.0)
    tot = jnp.zeros((1, NUM_EXPERTS), jnp.float32)
    parts = []
    for c in range(NCB):
        mc = m_both[c * CB:(c + 1) * CB]
        parts.append(jnp.dot(tri, mc, preferred_element_type=jnp.float32)
                     + tot)
        tot = tot + jnp.sum(mc, axis=0, keepdims=True)
    s_excl = jnp.concatenate(parts, axis=0)
    counts = tot

    nb = jnp.floor((counts + (BT - 1)) * (1.0 / BT))
    r8 = lax.broadcasted_iota(jnp.int32, (NUM_EXPERTS, NUM_EXPERTS), 0)
    c8 = lax.broadcasted_iota(jnp.int32, (NUM_EXPERTS, NUM_EXPERTS), 1)
    lt8 = jnp.where(r8 < c8, 1.0, 0.0)
    bounds = jnp.dot(nb, lt8, preferred_element_type=jnp.float32)
    pad_base = bounds * float(BT)

    pos0 = (jnp.sum(oh1 * pad_base, axis=1, keepdims=True)
            + jnp.sum(oh1 * s_excl, axis=1, keepdims=True))
    pos1 = (jnp.sum(oh2 * pad_base, axis=1, keepdims=True)
            + jnp.sum(oh2 * s_excl, axis=1, keepdims=True))
    pos0_ref[...] = pos0.astype(jnp.int32)
    pos1_ref[...] = pos1.astype(jnp.int32)
    pw0_ref[...] = jnp.broadcast_to(p1, (T, 16))
    pw1_ref[...] = jnp.broadcast_to(p2, (T, 16))

    gi = lax.broadcasted_iota(jnp.int32, (NG, NUM_EXPERTS), 0)
    ge = jnp.where(gi >= bounds.astype(jnp.int32), 1.0, 0.0)
    blk = jnp.sum(ge, axis=1, keepdims=True) - 1.0
    blk_ref[...] = jnp.clip(blk, 0.0, NUM_EXPERTS - 1).astype(jnp.int32)


def _ffn_kernel(blk_e_ref, xg_ref, w1_ref, b1_ref, w2_ref, b2_ref, y_ref):
    del blk_e_ref
    h = jnp.dot(xg_ref[...], w1_ref[0], preferred_element_type=jnp.float32)
    h = jnp.maximum(h + b1_ref[0], 0.0)
    y_ref[...] = (jnp.dot(h, w2_ref[0], preferred_element_type=jnp.float32)
                  + b2_ref[0])


def _dispatch_kernel(x_hbm, p0_hbm, p1_hbm, xg_hbm, p0_v, p1_v, xbuf,
                     gsem, wsem):
    wid = lax.axis_index("s") * 2 + lax.axis_index("c")
    base = wid * TPW
    pltpu.sync_copy(p0_hbm.at[pl.ds(base, TPW)], p0_v)
    pltpu.sync_copy(p1_hbm.at[pl.ds(base, TPW)], p1_v)
    pltpu.async_copy(x_hbm.at[pl.ds(base, TPW)], xbuf, gsem).wait()
    s0 = pltpu.async_copy(xbuf, xg_hbm.at[p0_v], wsem)
    s1 = pltpu.async_copy(xbuf, xg_hbm.at[p1_v], wsem)
    s0.wait()
    s1.wait()


def _combine_kernel(y_hbm, p0_hbm, p1_hbm, pw0_hbm, pw1_hbm, out_hbm,
                    p0_v, p1_v, w0_v, w1_v, buf0, buf1, sem):
    wid = lax.axis_index("s") * 2 + lax.axis_index("c")
    base = wid * TPW
    pltpu.sync_copy(p0_hbm.at[pl.ds(base, TPW)], p0_v)
    pltpu.sync_copy(p1_hbm.at[pl.ds(base, TPW)], p1_v)
    pltpu.sync_copy(pw0_hbm.at[pl.ds(base * 16, TPW * 16)], w0_v)
    pltpu.sync_copy(pw1_hbm.at[pl.ds(base * 16, TPW * 16)], w1_v)
    c0 = pltpu.async_copy(y_hbm.at[p0_v], buf0, sem)
    c1 = pltpu.async_copy(y_hbm.at[p1_v], buf1, sem)
    c0.wait()
    c1.wait()

    def row(r, carry):
        w0 = w0_v[pl.ds(r * 16, 16)]
        w1 = w1_v[pl.ds(r * 16, 16)]
        for j in range(EMBED_DIM // 16):
            sl = pl.ds(j * 16, 16)
            buf0[r, sl] = w0 * buf0[r, sl] + w1 * buf1[r, sl]
        return carry

    lax.fori_loop(0, TPW, row, 0)
    pltpu.sync_copy(buf0, out_hbm.at[pl.ds(base, TPW)])


def kernel(x, Wr, br, W1, b1, W2, b2):
    batch, seq, d = x.shape
    x_flat = x.reshape(-1, d)
    T = x_flat.shape[0]

    pos0, pos1, pw0b, pw1b, blk_e = pl.pallas_call(
        _router_kernel,
        grid=(1,),
        in_specs=[
            pl.BlockSpec((T, d), lambda i: (0, 0)),
            pl.BlockSpec((d, NUM_EXPERTS), lambda i: (0, 0)),
            pl.BlockSpec((1, NUM_EXPERTS), lambda i: (0, 0)),
        ],
        out_specs=[
            pl.BlockSpec((T, 1), lambda i: (0, 0)),
            pl.BlockSpec((T, 1), lambda i: (0, 0)),
            pl.BlockSpec((T, 16), lambda i: (0, 0)),
            pl.BlockSpec((T, 16), lambda i: (0, 0)),
            pl.BlockSpec((NG, 1), lambda i: (0, 0)),
        ],
        out_shape=[
            jax.ShapeDtypeStruct((T, 1), jnp.int32),
            jax.ShapeDtypeStruct((T, 1), jnp.int32),
            jax.ShapeDtypeStruct((T, 16), jnp.float32),
            jax.ShapeDtypeStruct((T, 16), jnp.float32),
            jax.ShapeDtypeStruct((NG, 1), jnp.int32),
        ],
    )(x_flat, Wr, br.reshape(1, NUM_EXPERTS))

    pos0 = pos0.reshape(T)
    pos1 = pos1.reshape(T)
    pw0b = pw0b.reshape(T * 16)
    pw1b = pw1b.reshape(T * 16)
    blk_e = blk_e.reshape(NG)

    mesh = plsc.VectorSubcoreMesh(core_axis_name="c", subcore_axis_name="s")

    xg = pl.kernel(
        _dispatch_kernel,
        mesh=mesh,
        out_type=jax.ShapeDtypeStruct((NP_ROWS, d), jnp.float32),
        scratch_types=[
            pltpu.VMEM((TPW,), jnp.int32),
            pltpu.VMEM((TPW,), jnp.int32),
            pltpu.VMEM((TPW, d), jnp.float32),
            pltpu.SemaphoreType.DMA,
            pltpu.SemaphoreType.DMA,
        ],
    )(x_flat, pos0, pos1)

    y = pl.pallas_call(
        _ffn_kernel,
        grid_spec=pltpu.PrefetchScalarGridSpec(
            num_scalar_prefetch=1,
            grid=(NG,),
            in_specs=[
                pl.BlockSpec((BT, d), lambda g, be: (g, 0)),
                pl.BlockSpec((1, d, HIDDEN_DIM), lambda g, be: (be[g], 0, 0)),
                pl.BlockSpec((1, 1, HIDDEN_DIM), lambda g, be: (be[g], 0, 0)),
                pl.BlockSpec((1, HIDDEN_DIM, d), lambda g, be: (be[g], 0, 0)),
                pl.BlockSpec((1, 1, d), lambda g, be: (be[g], 0, 0)),
            ],
            out_specs=pl.BlockSpec((BT, d), lambda g, be: (g, 0)),
        ),
        out_shape=jax.ShapeDtypeStruct((NP_ROWS, d), jnp.float32),
    )(blk_e, xg, W1,
      b1.reshape(NUM_EXPERTS, 1, HIDDEN_DIM), W2,
      b2.reshape(NUM_EXPERTS, 1, EMBED_DIM))

    out = pl.kernel(
        _combine_kernel,
        mesh=mesh,
        out_type=jax.ShapeDtypeStruct((T, d), jnp.float32),
        scratch_types=[
            pltpu.VMEM((TPW,), jnp.int32),
            pltpu.VMEM((TPW,), jnp.int32),
            pltpu.VMEM((TPW * 16,), jnp.float32),
            pltpu.VMEM((TPW * 16,), jnp.float32),
            pltpu.VMEM((TPW, d), jnp.float32),
            pltpu.VMEM((TPW, d), jnp.float32),
            pltpu.SemaphoreType.DMA,
        ],
    )(y, pos0, pos1, pw0b, pw1b)

    return out.reshape(batch, seq, d)


# speedup vs baseline: 1.8960x; 1.0005x over previous
import jax
import jax.numpy as jnp
from jax import lax
from jax.experimental import pallas as pl
from jax.experimental.pallas import tpu as pltpu
from jax.experimental.pallas import tpu_sc as plsc

EMBED_DIM = 768
HIDDEN_DIM = 3072
NUM_EXPERTS = 8
TOP_K = 2

BT = 512
T_TOKENS = 2048
NPAIR = T_TOKENS * TOP_K
NG = NPAIR // BT + NUM_EXPERTS
NP_ROWS = NG * BT

NW = 32
TPW = T_TOKENS // NW

CB = 128
NCB = T_TOKENS // CB


def _router_kernel(x_ref, wr_ref, br_ref,
                   pos0_ref, pos1_ref, pw0_ref, pw1_ref, blk_ref):
    T = T_TOKENS
    logits = jnp.dot(x_ref[...], wr_ref[...],
                     preferred_element_type=jnp.float32) + br_ref[...]
    lane = lax.broadcasted_iota(jnp.int32, logits.shape, 1)
    m1 = jnp.max(logits, axis=1, keepdims=True)
    i1 = jnp.min(jnp.where(logits == m1, lane, NUM_EXPERTS), axis=1,
                 keepdims=True)
    l2 = jnp.where(lane == i1, -jnp.inf, logits)
    m2 = jnp.max(l2, axis=1, keepdims=True)
    i2 = jnp.min(jnp.where(l2 == m2, lane, NUM_EXPERTS), axis=1, keepdims=True)
    p2 = 1.0 / (1.0 + jnp.exp(m1 - m2))
    p1 = 1.0 - p2

    oh1 = jnp.where(lane == i1, 1.0, 0.0)
    oh2 = jnp.where(lane == i2, 1.0, 0.0)
    m_both = oh1 + oh2

    r_i = lax.broadcasted_iota(jnp.int32, (CB, CB), 0)
    c_i = lax.broadcasted_iota(jnp.int32, (CB, CB), 1)
    tri = jnp.where(r_i > c_i, 1.0, 0.0)
    tot = jnp.zeros((1, NUM_EXPERTS), jnp.float32)
    parts = []
    for c in range(NCB):
        mc = m_both[c * CB:(c + 1) * CB]
        parts.append(jnp.dot(tri, mc, preferred_element_type=jnp.float32)
                     + tot)
        tot = tot + jnp.sum(mc, axis=0, keepdims=True)
    s_excl = jnp.concatenate(parts, axis=0)
    counts = tot

    nb = jnp.floor((counts + (BT - 1)) * (1.0 / BT))
    r8 = lax.broadcasted_iota(jnp.int32, (NUM_EXPERTS, NUM_EXPERTS), 0)
    c8 = lax.broadcasted_iota(jnp.int32, (NUM_EXPERTS, NUM_EXPERTS), 1)
    lt8 = jnp.where(r8 < c8, 1.0, 0.0)
    bounds = jnp.dot(nb, lt8, preferred_element_type=jnp.float32)
    pad_base = bounds * float(BT)

    pos0 = (jnp.sum(oh1 * pad_base, axis=1, keepdims=True)
            + jnp.sum(oh1 * s_excl, axis=1, keepdims=True))
    pos1 = (jnp.sum(oh2 * pad_base, axis=1, keepdims=True)
            + jnp.sum(oh2 * s_excl, axis=1, keepdims=True))
    pos0_ref[...] = pos0.astype(jnp.int32)
    pos1_ref[...] = pos1.astype(jnp.int32)
    pw0_ref[...] = jnp.broadcast_to(p1, (T, 16))
    pw1_ref[...] = jnp.broadcast_to(p2, (T, 16))

    gi = lax.broadcasted_iota(jnp.int32, (NG, NUM_EXPERTS), 0)
    ge = jnp.where(gi >= bounds.astype(jnp.int32), 1.0, 0.0)
    blk = jnp.sum(ge, axis=1, keepdims=True) - 1.0
    blk_ref[...] = jnp.clip(blk, 0.0, NUM_EXPERTS - 1).astype(jnp.int32)


def _ffn_kernel(blk_e_ref, xg_ref, w1_ref, b1_ref, w2_ref, b2_ref, y_ref):
    del blk_e_ref
    h = jnp.dot(xg_ref[...], w1_ref[0], preferred_element_type=jnp.float32)
    h = jnp.maximum(h + b1_ref[0], 0.0)
    y_ref[...] = (jnp.dot(h, w2_ref[0], preferred_element_type=jnp.float32)
                  + b2_ref[0])


def _dispatch_kernel(x_hbm, p0_hbm, p1_hbm, xg_hbm, p0_v, p1_v, xbuf,
                     gsem, wsem):
    wid = lax.axis_index("s") * 2 + lax.axis_index("c")
    base = wid * TPW
    pltpu.sync_copy(p0_hbm.at[pl.ds(base, TPW)], p0_v)
    pltpu.sync_copy(p1_hbm.at[pl.ds(base, TPW)], p1_v)
    pltpu.async_copy(x_hbm.at[pl.ds(base, TPW)], xbuf, gsem).wait()
    s0 = pltpu.async_copy(xbuf, xg_hbm.at[p0_v], wsem)
    s1 = pltpu.async_copy(xbuf, xg_hbm.at[p1_v], wsem)
    s0.wait()
    s1.wait()


def _combine_kernel(y_hbm, p0_hbm, p1_hbm, pw0_hbm, pw1_hbm, out_hbm,
                    p0_v, p1_v, w0_v, w1_v, buf0, buf1, sem):
    wid = lax.axis_index("s") * 2 + lax.axis_index("c")
    base = wid * TPW
    pltpu.sync_copy(p0_hbm.at[pl.ds(base, TPW)], p0_v)
    pltpu.sync_copy(p1_hbm.at[pl.ds(base, TPW)], p1_v)
    pltpu.sync_copy(pw0_hbm.at[pl.ds(base * 16, TPW * 16)], w0_v)
    pltpu.sync_copy(pw1_hbm.at[pl.ds(base * 16, TPW * 16)], w1_v)
    c0 = pltpu.async_copy(y_hbm.at[p0_v], buf0, sem)
    c1 = pltpu.async_copy(y_hbm.at[p1_v], buf1, sem)
    c0.wait()
    c1.wait()

    def row(r, carry):
        w0 = w0_v[pl.ds(r * 16, 16)]
        w1 = w1_v[pl.ds(r * 16, 16)]
        for j in range(EMBED_DIM // 16):
            sl = pl.ds(j * 16, 16)
            buf0[r, sl] = w0 * buf0[r, sl] + w1 * buf1[r, sl]
        return carry

    lax.fori_loop(0, TPW, row, 0)
    pltpu.sync_copy(buf0, out_hbm.at[pl.ds(base, TPW)])


def kernel(x, Wr, br, W1, b1, W2, b2):
    batch, seq, d = x.shape
    x_flat = x.reshape(-1, d)
    T = x_flat.shape[0]

    pos0, pos1, pw0b, pw1b, blk_e = pl.pallas_call(
        _router_kernel,
        grid=(1,),
        in_specs=[
            pl.BlockSpec((T, d), lambda i: (0, 0)),
            pl.BlockSpec((d, NUM_EXPERTS), lambda i: (0, 0)),
            pl.BlockSpec((1, NUM_EXPERTS), lambda i: (0, 0)),
        ],
        out_specs=[
            pl.BlockSpec((T, 1), lambda i: (0, 0)),
            pl.BlockSpec((T, 1), lambda i: (0, 0)),
            pl.BlockSpec((T, 16), lambda i: (0, 0)),
            pl.BlockSpec((T, 16), lambda i: (0, 0)),
            pl.BlockSpec((NG, 1), lambda i: (0, 0)),
        ],
        out_shape=[
            jax.ShapeDtypeStruct((T, 1), jnp.int32),
            jax.ShapeDtypeStruct((T, 1), jnp.int32),
            jax.ShapeDtypeStruct((T, 16), jnp.float32),
            jax.ShapeDtypeStruct((T, 16), jnp.float32),
            jax.ShapeDtypeStruct((NG, 1), jnp.int32),
        ],
    )(x_flat, Wr, br.reshape(1, NUM_EXPERTS))

    pos0 = pos0.reshape(T)
    pos1 = pos1.reshape(T)
    pw0b = pw0b.reshape(T * 16)
    pw1b = pw1b.reshape(T * 16)
    blk_e = blk_e.reshape(NG)

    mesh = plsc.VectorSubcoreMesh(core_axis_name="c", subcore_axis_name="s")

    xg = pl.kernel(
        _dispatch_kernel,
        mesh=mesh,
        out_type=jax.ShapeDtypeStruct((NP_ROWS, d), jnp.float32),
        scratch_types=[
            pltpu.VMEM((TPW,), jnp.int32),
            pltpu.VMEM((TPW,), jnp.int32),
            pltpu.VMEM((TPW, d), jnp.float32),
            pltpu.SemaphoreType.DMA,
            pltpu.SemaphoreType.DMA,
        ],
    )(x_flat, pos0, pos1)

    y = pl.pallas_call(
        _ffn_kernel,
        grid_spec=pltpu.PrefetchScalarGridSpec(
            num_scalar_prefetch=1,
            grid=(NG,),
            in_specs=[
                pl.BlockSpec((BT, d), lambda g, be: (g, 0)),
                pl.BlockSpec((1, d, HIDDEN_DIM), lambda g, be: (be[g], 0, 0)),
                pl.BlockSpec((1, 1, HIDDEN_DIM), lambda g, be: (be[g], 0, 0)),
                pl.BlockSpec((1, HIDDEN_DIM, d), lambda g, be: (be[g], 0, 0)),
                pl.BlockSpec((1, 1, d), lambda g, be: (be[g], 0, 0)),
            ],
            out_specs=pl.BlockSpec((BT, d), lambda g, be: (g, 0)),
        ),
        out_shape=jax.ShapeDtypeStruct((NP_ROWS, d), jnp.float32),
    )(blk_e, xg, W1,
      b1.reshape(NUM_EXPERTS, 1, HIDDEN_DIM), W2,
      b2.reshape(NUM_EXPERTS, 1, EMBED_DIM))

    out = pl.kernel(
        _combine_kernel,
        mesh=mesh,
        out_type=jax.ShapeDtypeStruct((T, d), jnp.float32),
        scratch_types=[
            pltpu.VMEM((TPW,), jnp.int32),
            pltpu.VMEM((TPW,), jnp.int32),
            pltpu.VMEM((TPW * 16,), jnp.float32),
            pltpu.VMEM((TPW * 16,), jnp.float32),
            pltpu.VMEM((TPW, d), jnp.float32),
            pltpu.VMEM((TPW, d), jnp.float32),
            pltpu.SemaphoreType.DMA,
        ],
    )(y, pos0, pos1, pw0b, pw1b)

    return out.reshape(batch, seq, d)
